# Initial kernel scaffold; baseline (speedup 1.0000x reference)
import jax
import jax.numpy as jnp
from jax.experimental import pallas as pl


def kernel(nodes, g, ratings, row_idxs, col_idxs, features, rating_emb, W1, b1, W2, b2):
    raise NotImplementedError("write your pallas kernel here")


# speedup vs baseline: 3.2545x; 3.2545x over previous
import functools

import jax
import jax.numpy as jnp
from jax import lax
from jax.experimental import pallas as pl
from jax.experimental.pallas import tpu as pltpu
from jax.experimental.pallas import tpu_sc as plsc


def _sc_gather(features, col2d, nodes2d, E, B, D):
    NW = 32
    IPC = 128
    epw = E // NW
    bpw = B // NW
    ech = epw // IPC
    mesh = plsc.VectorSubcoreMesh(core_axis_name="c", subcore_axis_name="s")

    @functools.partial(
        pl.kernel,
        mesh=mesh,
        out_type=(
            jax.ShapeDtypeStruct((E, D), jnp.float32),
            jax.ShapeDtypeStruct((B, D), jnp.float32),
        ),
        scratch_types=[
            pltpu.VMEM((ech, IPC), jnp.int32),
            pltpu.VMEM((bpw // IPC if bpw >= IPC else 1, IPC), jnp.int32),
            pltpu.VMEM((IPC, D), jnp.float32),
            pltpu.SemaphoreType.DMA,
        ],
    )
    def k(feat_hbm, col_hbm, nodes_hbm, g_out, s_out, colv, nodev, rows, sem):
        wid = lax.axis_index("s") * 2 + lax.axis_index("c")
        base = wid * epw
        pltpu.sync_copy(col_hbm.at[pl.ds(wid * ech, ech)], colv)

        def body(j, carry):
            pltpu.async_copy(feat_hbm.at[colv.at[j]], rows, sem).wait()
            pltpu.sync_copy(rows, g_out.at[pl.ds(base + j * IPC, IPC)])
            return carry

        lax.fori_loop(0, ech, body, 0)

        nbase = wid * bpw
        pltpu.sync_copy(nodes_hbm.at[pl.ds(wid, 1)], nodev)
        pltpu.async_copy(feat_hbm.at[nodev.at[0]], rows, sem).wait()
        pltpu.sync_copy(rows, s_out.at[pl.ds(nbase, IPC)])

    return k(features, col2d, nodes2d)


def _tc_body(G_any, rowcol_any, ratcol_any, rowfull_ref, S_ref, remb_ref,
             W1a_ref, W1b_ref, b1_ref, W2a_ref, W2b_ref, b2_ref,
             out_ref, euv, rowc, ratc, sem0, sem1, sem2,
             *, RG, CH, D):
    g_idx = pl.program_id(0)
    base_row = g_idx * RG

    rowfull = rowfull_ref[...]
    s = jnp.sum((rowfull < base_row).astype(jnp.int32))
    e = jnp.sum((rowfull < base_row + RG).astype(jnp.int32))
    k0 = s // CH
    k1 = (e + CH - 1) // CH

    rproj = jnp.dot(remb_ref[...], W1b_ref[...],
                    preferred_element_type=jnp.float32) + b1_ref[...]

    iota_rg = lax.broadcasted_iota(jnp.int32, (CH, RG), 1)
    iota_r8 = lax.broadcasted_iota(jnp.int32, (CH, 8), 1)
    ones_c8 = jnp.ones((CH, 8), jnp.float32)

    def chunk(k, carry):
        sums, counts = carry
        c1 = pltpu.make_async_copy(G_any.at[pl.ds(k * CH, CH)], euv, sem0)
        c2 = pltpu.make_async_copy(rowcol_any.at[pl.ds(k * CH, CH)], rowc, sem1)
        c3 = pltpu.make_async_copy(ratcol_any.at[pl.ds(k * CH, CH)], ratc, sem2)
        c1.start()
        c2.start()
        c3.start()
        c1.wait()
        c2.wait()
        c3.wait()
        M = ((rowc[...] - base_row) == iota_rg).astype(jnp.float32)
        oh_r = (ratc[...] == iota_r8).astype(jnp.float32)
        re = jnp.dot(oh_r, rproj, preferred_element_type=jnp.float32)
        x = jnp.maximum(
            jnp.dot(euv[...], W1a_ref[...],
                    preferred_element_type=jnp.float32) + re, 0.0)
        sums = sums + lax.dot_general(
            M, x, (((0,), (0,)), ((), ())), preferred_element_type=jnp.float32)
        counts = counts + lax.dot_general(
            M, ones_c8, (((0,), (0,)), ((), ())),
            preferred_element_type=jnp.float32)
        return sums, counts

    sums0 = jnp.zeros((RG, D), jnp.float32)
    counts0 = jnp.zeros((RG, 8), jnp.float32)
    sums, counts = lax.fori_loop(k0, k1, chunk, (sums0, counts0))

    neigh = sums / jnp.maximum(counts[:, 0:1], 1.0)
    out = (jnp.dot(S_ref[...], W2a_ref[...], preferred_element_type=jnp.float32)
           + jnp.dot(neigh, W2b_ref[...], preferred_element_type=jnp.float32)
           + b2_ref[...])
    out_ref[...] = jnp.maximum(out, 0.0)


def _tc_fused(G, S, rowfull, rowcol, ratcol, remb8, W1a, W1b, b1,
              W2a, W2b, b2, E, B, D, interpret=False):
    RG = 128
    CH = 1024
    grid = (B // RG,)

    return pl.pallas_call(
        functools.partial(_tc_body, RG=RG, CH=CH, D=D),
        grid=grid,
        in_specs=[
            pl.BlockSpec(memory_space=pl.ANY),
            pl.BlockSpec(memory_space=pl.ANY),
            pl.BlockSpec(memory_space=pl.ANY),
            pl.BlockSpec((E // 128, 128), lambda i: (0, 0)),
            pl.BlockSpec((RG, D), lambda i: (i, 0)),
            pl.BlockSpec((8, D), lambda i: (0, 0)),
            pl.BlockSpec((D, D), lambda i: (0, 0)),
            pl.BlockSpec((D, D), lambda i: (0, 0)),
            pl.BlockSpec((1, D), lambda i: (0, 0)),
            pl.BlockSpec((D, D), lambda i: (0, 0)),
            pl.BlockSpec((D, D), lambda i: (0, 0)),
            pl.BlockSpec((1, D), lambda i: (0, 0)),
        ],
        out_specs=pl.BlockSpec((RG, D), lambda i: (i, 0)),
        out_shape=jax.ShapeDtypeStruct((B, D), jnp.float32),
        scratch_shapes=[
            pltpu.VMEM((CH, D), jnp.float32),
            pltpu.VMEM((CH, 1), jnp.int32),
            pltpu.VMEM((CH, 1), jnp.int32),
            pltpu.SemaphoreType.DMA,
            pltpu.SemaphoreType.DMA,
            pltpu.SemaphoreType.DMA,
        ],
        compiler_params=pltpu.CompilerParams(
            dimension_semantics=("arbitrary",)),
        interpret=interpret,
    )(G, rowcol, ratcol, rowfull, S, remb8, W1a, W1b, b1, W2a, W2b, b2)


def kernel(nodes, g, ratings, row_idxs, col_idxs, features, rating_emb,
           W1, b1, W2, b2):
    E = col_idxs.shape[0]
    B = nodes.shape[0]
    V, D = features.shape
    R = rating_emb.shape[0]

    col2d = col_idxs.reshape(E // 128, 128)
    nodes2d = nodes.reshape(B // 128, 128)
    G, S = _sc_gather(features, col2d, nodes2d, E, B, D)

    rowfull = row_idxs.reshape(E // 128, 128)
    rowcol = row_idxs.reshape(E, 1)
    ratcol = ratings.reshape(E, 1)
    remb8 = jnp.zeros((8, D), jnp.float32).at[:R].set(rating_emb)
    W1a, W1b = W1[:D], W1[D:]
    W2a, W2b = W2[:D], W2[D:]
    return _tc_fused(G, S, rowfull, rowcol, ratcol, remb8,
                     W1a, W1b, b1.reshape(1, D),
                     W2a, W2b, b2.reshape(1, D), E, B, D)


# speedup vs baseline: 4.4716x; 1.3740x over previous
import functools

import jax
import jax.numpy as jnp
from jax import lax
from jax.experimental import pallas as pl
from jax.experimental.pallas import tpu as pltpu
from jax.experimental.pallas import tpu_sc as plsc


def _sc_gather(features, col2d, nodes2d, E, B, D):
    NW = 32
    IPC = 128
    epw = E // NW
    bpw = B // NW
    ech = epw // IPC
    mesh = plsc.VectorSubcoreMesh(core_axis_name="c", subcore_axis_name="s")

    @functools.partial(
        pl.kernel,
        mesh=mesh,
        out_type=(
            jax.ShapeDtypeStruct((E, D), jnp.float32),
            jax.ShapeDtypeStruct((B, D), jnp.float32),
        ),
        scratch_types=[
            pltpu.VMEM((ech, IPC), jnp.int32),
            pltpu.VMEM((bpw // IPC if bpw >= IPC else 1, IPC), jnp.int32),
            pltpu.VMEM((IPC, D), jnp.float32),
            pltpu.VMEM((IPC, D), jnp.float32),
            pltpu.SemaphoreType.DMA,
            pltpu.SemaphoreType.DMA,
        ],
    )
    def k(feat_hbm, col_hbm, nodes_hbm, g_out, s_out, colv, nodev,
          rows_a, rows_b, sem_a, sem_b):
        wid = lax.axis_index("s") * 2 + lax.axis_index("c")
        base = wid * epw
        pltpu.sync_copy(col_hbm.at[pl.ds(wid * ech, ech)], colv)

        def start(j, buf, sem):
            pltpu.make_async_copy(feat_hbm.at[colv.at[j]], buf, sem).start()

        def finish(j, buf, sem):
            pltpu.make_async_copy(feat_hbm.at[colv.at[j]], buf, sem).wait()
            pltpu.sync_copy(buf, g_out.at[pl.ds(base + j * IPC, IPC)])

        start(0, rows_a, sem_a)

        def body(p, carry):
            j0 = 2 * p
            start(j0 + 1, rows_b, sem_b)
            finish(j0, rows_a, sem_a)

            @pl.when(j0 + 2 < ech)
            def _():
                start(j0 + 2, rows_a, sem_a)

            finish(j0 + 1, rows_b, sem_b)
            return carry

        lax.fori_loop(0, ech // 2, body, 0)

        nbase = wid * bpw
        pltpu.sync_copy(nodes_hbm.at[pl.ds(wid, 1)], nodev)
        pltpu.async_copy(feat_hbm.at[nodev.at[0]], rows_a, sem_a).wait()
        pltpu.sync_copy(rows_a, s_out.at[pl.ds(nbase, IPC)])

    return k(features, col2d, nodes2d)


def _tc_body(G_any, rowcol_any, ratcol_any, rowfull_ref, S_ref, remb_ref,
             W1a_ref, W1b_ref, b1_ref, W2a_ref, W2b_ref, b2_ref,
             out_ref, euv2, rowc2, ratc2, semg, semr, semt,
             *, RG, CH, D):
    g_idx = pl.program_id(0)
    base_row = g_idx * RG

    rowfull = rowfull_ref[...]
    s = jnp.sum((rowfull < base_row).astype(jnp.int32))
    e = jnp.sum((rowfull < base_row + RG).astype(jnp.int32))
    k0 = s // CH
    k1 = (e + CH - 1) // CH
    n = k1 - k0

    rproj = (jnp.dot(remb_ref[...], W1b_ref[...],
                     preferred_element_type=jnp.float32)
             + b1_ref[...]).astype(jnp.bfloat16)
    w1a = W1a_ref[...].astype(jnp.bfloat16)

    iota_rg = lax.broadcasted_iota(jnp.int32, (CH, RG), 1)
    iota_r8 = lax.broadcasted_iota(jnp.int32, (CH, 8), 1)
    ones_c8 = jnp.ones((CH, 8), jnp.bfloat16)

    def copies(k, slot):
        return (
            pltpu.make_async_copy(G_any.at[pl.ds(k * CH, CH)],
                                  euv2.at[slot], semg.at[slot]),
            pltpu.make_async_copy(rowcol_any.at[pl.ds(k * CH, CH)],
                                  rowc2.at[slot], semr.at[slot]),
            pltpu.make_async_copy(ratcol_any.at[pl.ds(k * CH, CH)],
                                  ratc2.at[slot], semt.at[slot]),
        )

    def issue(k, slot):
        for c in copies(k, slot):
            c.start()

    @pl.when(n > 0)
    def _():
        issue(k0, 0)

    def chunk(i, carry):
        sums, counts = carry
        k = k0 + i
        slot = lax.rem(i, 2)

        @pl.when(i + 1 < n)
        def _():
            issue(k + 1, 1 - slot)

        for c in copies(k, slot):
            c.wait()

        M = ((rowc2[slot] - base_row) == iota_rg).astype(jnp.bfloat16)
        oh_r = (ratc2[slot] == iota_r8).astype(jnp.bfloat16)
        re = jnp.dot(oh_r, rproj, preferred_element_type=jnp.float32)
        x = jnp.maximum(
            jnp.dot(euv2[slot].astype(jnp.bfloat16), w1a,
                    preferred_element_type=jnp.float32) + re,
            0.0).astype(jnp.bfloat16)
        sums = sums + lax.dot_general(
            M, x, (((0,), (0,)), ((), ())), preferred_element_type=jnp.float32)
        counts = counts + lax.dot_general(
            M, ones_c8, (((0,), (0,)), ((), ())),
            preferred_element_type=jnp.float32)
        return sums, counts

    sums0 = jnp.zeros((RG, D), jnp.float32)
    counts0 = jnp.zeros((RG, 8), jnp.float32)
    sums, counts = lax.fori_loop(0, n, chunk, (sums0, counts0))

    neigh = sums / jnp.maximum(counts[:, 0:1], 1.0)
    out = (jnp.dot(S_ref[...], W2a_ref[...], preferred_element_type=jnp.float32)
           + jnp.dot(neigh, W2b_ref[...], preferred_element_type=jnp.float32)
           + b2_ref[...])
    out_ref[...] = jnp.maximum(out, 0.0)


def _tc_fused(G, S, rowfull, rowcol, ratcol, remb8, W1a, W1b, b1,
              W2a, W2b, b2, E, B, D, interpret=False):
    RG = 128
    CH = 1024
    grid = (B // RG,)

    return pl.pallas_call(
        functools.partial(_tc_body, RG=RG, CH=CH, D=D),
        grid=grid,
        in_specs=[
            pl.BlockSpec(memory_space=pl.ANY),
            pl.BlockSpec(memory_space=pl.ANY),
            pl.BlockSpec(memory_space=pl.ANY),
            pl.BlockSpec((E // 128, 128), lambda i: (0, 0)),
            pl.BlockSpec((RG, D), lambda i: (i, 0)),
            pl.BlockSpec((8, D), lambda i: (0, 0)),
            pl.BlockSpec((D, D), lambda i: (0, 0)),
            pl.BlockSpec((D, D), lambda i: (0, 0)),
            pl.BlockSpec((1, D), lambda i: (0, 0)),
            pl.BlockSpec((D, D), lambda i: (0, 0)),
            pl.BlockSpec((D, D), lambda i: (0, 0)),
            pl.BlockSpec((1, D), lambda i: (0, 0)),
        ],
        out_specs=pl.BlockSpec((RG, D), lambda i: (i, 0)),
        out_shape=jax.ShapeDtypeStruct((B, D), jnp.float32),
        scratch_shapes=[
            pltpu.VMEM((2, CH, D), jnp.float32),
            pltpu.VMEM((2, CH, 1), jnp.int32),
            pltpu.VMEM((2, CH, 1), jnp.int32),
            pltpu.SemaphoreType.DMA((2,)),
            pltpu.SemaphoreType.DMA((2,)),
            pltpu.SemaphoreType.DMA((2,)),
        ],
        compiler_params=pltpu.CompilerParams(
            dimension_semantics=("arbitrary",)),
        interpret=interpret,
    )(G, rowcol, ratcol, rowfull, S, remb8, W1a, W1b, b1, W2a, W2b, b2)


def kernel(nodes, g, ratings, row_idxs, col_idxs, features, rating_emb,
           W1, b1, W2, b2):
    E = col_idxs.shape[0]
    B = nodes.shape[0]
    V, D = features.shape
    R = rating_emb.shape[0]

    col2d = col_idxs.reshape(E // 128, 128)
    nodes2d = nodes.reshape(B // 128, 128)
    G, S = _sc_gather(features, col2d, nodes2d, E, B, D)

    rowfull = row_idxs.reshape(E // 128, 128)
    rowcol = row_idxs.reshape(E, 1)
    ratcol = ratings.reshape(E, 1)
    remb8 = jnp.zeros((8, D), jnp.float32).at[:R].set(rating_emb)
    W1a, W1b = W1[:D], W1[D:]
    W2a, W2b = W2[:D], W2[D:]
    return _tc_fused(G, S, rowfull, rowcol, ratcol, remb8,
                     W1a, W1b, b1.reshape(1, D),
                     W2a, W2b, b2.reshape(1, D), E, B, D)


# speedup vs baseline: 5.9184x; 1.3236x over previous
import functools

import jax
import jax.numpy as jnp
from jax import lax
from jax.experimental import pallas as pl
from jax.experimental.pallas import tpu as pltpu
from jax.experimental.pallas import tpu_sc as plsc


def _sc_gather(features, col2d, nodes2d, E, B, D):
    NW = 32
    IPC = 128
    epw = E // NW
    bpw = B // NW
    ech = epw // IPC
    mesh = plsc.VectorSubcoreMesh(core_axis_name="c", subcore_axis_name="s")

    @functools.partial(
        pl.kernel,
        mesh=mesh,
        out_type=(
            jax.ShapeDtypeStruct((E, D), jnp.float32),
            jax.ShapeDtypeStruct((B, D), jnp.float32),
        ),
        scratch_types=[
            pltpu.VMEM((ech, IPC), jnp.int32),
            pltpu.VMEM((bpw // IPC if bpw >= IPC else 1, IPC), jnp.int32),
            pltpu.VMEM((IPC, D), jnp.float32),
            pltpu.VMEM((IPC, D), jnp.float32),
            pltpu.SemaphoreType.DMA,
            pltpu.SemaphoreType.DMA,
        ],
    )
    def k(feat_hbm, col_hbm, nodes_hbm, g_out, s_out, colv, nodev,
          rows_a, rows_b, sem_a, sem_b):
        wid = lax.axis_index("s") * 2 + lax.axis_index("c")
        base = wid * epw
        pltpu.sync_copy(col_hbm.at[pl.ds(wid * ech, ech)], colv)

        def start(j, buf, sem):
            pltpu.make_async_copy(feat_hbm.at[colv.at[j]], buf, sem).start()

        def finish(j, buf, sem):
            pltpu.make_async_copy(feat_hbm.at[colv.at[j]], buf, sem).wait()
            pltpu.sync_copy(buf, g_out.at[pl.ds(base + j * IPC, IPC)])

        start(0, rows_a, sem_a)

        def body(p, carry):
            j0 = 2 * p
            start(j0 + 1, rows_b, sem_b)
            finish(j0, rows_a, sem_a)

            @pl.when(j0 + 2 < ech)
            def _():
                start(j0 + 2, rows_a, sem_a)

            finish(j0 + 1, rows_b, sem_b)
            return carry

        lax.fori_loop(0, ech // 2, body, 0)

        nbase = wid * bpw
        pltpu.sync_copy(nodes_hbm.at[pl.ds(wid, 1)], nodev)
        pltpu.async_copy(feat_hbm.at[nodev.at[0]], rows_a, sem_a).wait()
        pltpu.sync_copy(rows_a, s_out.at[pl.ds(nbase, IPC)])

    return k(features, col2d, nodes2d)


def _tc_body(G_any, rowrat_any, rowfull_ref, S_ref, remb_ref,
             W1a_ref, W1b_ref, b1_ref, W2a_ref, W2b_ref, b2_ref,
             out_ref, euv2, rr2, semg, semr,
             *, RG, CH, D):
    g_idx = pl.program_id(0)
    base_row = g_idx * RG

    rowfull = rowfull_ref[...]
    s = jnp.sum((rowfull < base_row).astype(jnp.int32))
    e = jnp.sum((rowfull < base_row + RG).astype(jnp.int32))
    k0 = s // CH
    k1 = (e + CH - 1) // CH
    n = k1 - k0

    rproj = (jnp.dot(remb_ref[...], W1b_ref[...],
                     preferred_element_type=jnp.float32)
             + b1_ref[...]).astype(jnp.bfloat16)
    w1a = W1a_ref[...].astype(jnp.bfloat16)

    iota_rg = lax.broadcasted_iota(jnp.int32, (RG, 1), 0)
    iota_r8 = lax.broadcasted_iota(jnp.int32, (8, 1), 0)
    ones_c8 = jnp.ones((CH, 8), jnp.bfloat16)

    def copies(k, slot):
        return (
            pltpu.make_async_copy(G_any.at[pl.ds(k * CH, CH)],
                                  euv2.at[slot], semg.at[slot]),
            pltpu.make_async_copy(rowrat_any.at[k], rr2.at[slot],
                                  semr.at[slot]),
        )

    def issue(k, slot):
        for c in copies(k, slot):
            c.start()

    @pl.when(n > 0)
    def _():
        issue(k0, 0)

    def chunk(i, carry):
        sums, counts = carry
        k = k0 + i
        slot = lax.rem(i, 2)

        @pl.when(i + 1 < n)
        def _():
            issue(k + 1, 1 - slot)

        for c in copies(k, slot):
            c.wait()

        rows = rr2[slot, 0:1, :]
        rats = rr2[slot, 1:2, :]
        Mt = ((rows - base_row) == iota_rg).astype(jnp.bfloat16)
        ohr_t = (rats == iota_r8).astype(jnp.bfloat16)
        re = lax.dot_general(ohr_t, rproj, (((0,), (0,)), ((), ())),
                             preferred_element_type=jnp.float32)
        x = jnp.maximum(
            jnp.dot(euv2[slot].astype(jnp.bfloat16), w1a,
                    preferred_element_type=jnp.float32) + re,
            0.0).astype(jnp.bfloat16)
        sums = sums + jnp.dot(Mt, x, preferred_element_type=jnp.float32)
        counts = counts + jnp.dot(Mt, ones_c8,
                                  preferred_element_type=jnp.float32)
        return sums, counts

    sums0 = jnp.zeros((RG, D), jnp.float32)
    counts0 = jnp.zeros((RG, 8), jnp.float32)
    sums, counts = lax.fori_loop(0, n, chunk, (sums0, counts0))

    neigh = sums / jnp.maximum(counts[:, 0:1], 1.0)
    out = (jnp.dot(S_ref[...], W2a_ref[...], preferred_element_type=jnp.float32)
           + jnp.dot(neigh, W2b_ref[...], preferred_element_type=jnp.float32)
           + b2_ref[...])
    out_ref[...] = jnp.maximum(out, 0.0)


def _tc_fused(G, S, rowfull, rowrat, remb8, W1a, W1b, b1,
              W2a, W2b, b2, E, B, D, CH, interpret=False):
    RG = 128
    grid = (B // RG,)

    return pl.pallas_call(
        functools.partial(_tc_body, RG=RG, CH=CH, D=D),
        grid=grid,
        in_specs=[
            pl.BlockSpec(memory_space=pl.ANY),
            pl.BlockSpec(memory_space=pl.ANY),
            pl.BlockSpec((E // 128, 128), lambda i: (0, 0)),
            pl.BlockSpec((RG, D), lambda i: (i, 0)),
            pl.BlockSpec((8, D), lambda i: (0, 0)),
            pl.BlockSpec((D, D), lambda i: (0, 0)),
            pl.BlockSpec((D, D), lambda i: (0, 0)),
            pl.BlockSpec((1, D), lambda i: (0, 0)),
            pl.BlockSpec((D, D), lambda i: (0, 0)),
            pl.BlockSpec((D, D), lambda i: (0, 0)),
            pl.BlockSpec((1, D), lambda i: (0, 0)),
        ],
        out_specs=pl.BlockSpec((RG, D), lambda i: (i, 0)),
        out_shape=jax.ShapeDtypeStruct((B, D), jnp.float32),
        scratch_shapes=[
            pltpu.VMEM((2, CH, D), jnp.float32),
            pltpu.VMEM((2, 2, CH), jnp.int32),
            pltpu.SemaphoreType.DMA((2,)),
            pltpu.SemaphoreType.DMA((2,)),
        ],
        compiler_params=pltpu.CompilerParams(
            dimension_semantics=("arbitrary",)),
        interpret=interpret,
    )(G, rowrat, rowfull, S, remb8, W1a, W1b, b1, W2a, W2b, b2)


def kernel(nodes, g, ratings, row_idxs, col_idxs, features, rating_emb,
           W1, b1, W2, b2):
    E = col_idxs.shape[0]
    B = nodes.shape[0]
    V, D = features.shape
    R = rating_emb.shape[0]

    col2d = col_idxs.reshape(E // 128, 128)
    nodes2d = nodes.reshape(B // 128, 128)
    G, S = _sc_gather(features, col2d, nodes2d, E, B, D)

    CH = 1024
    rowfull = row_idxs.reshape(E // 128, 128)
    rowrat = jnp.stack(
        [row_idxs.reshape(E // CH, CH), ratings.reshape(E // CH, CH)], axis=1)
    remb8 = jnp.zeros((8, D), jnp.float32).at[:R].set(rating_emb)
    W1a, W1b = W1[:D], W1[D:]
    W2a, W2b = W2[:D], W2[D:]
    return _tc_fused(G, S, rowfull, rowrat, remb8,
                     W1a, W1b, b1.reshape(1, D),
                     W2a, W2b, b2.reshape(1, D), E, B, D, CH)
